# baseline (speedup 1.0000x reference)
import jax
import jax.numpy as jnp
from jax import lax
from jax.experimental import pallas as pl
from jax.experimental.pallas import tpu as pltpu
from jax.experimental.pallas import tpu_sc as plsc

N = 100000
D = 128
B = 16384
HALF_LIFE = 40.0
LN2 = 0.69314718

NC = 2
NS = 16
NW = NC * NS
RW = 3128
RWL = N - (NW - 1) * RW
TL = RW - RWL
PREV = 3136
CH = 128
CAP = 3200
NWV = PREV // 16
COEF = -LN2 / HALF_LIFE
UNR = 4


def _body(mem_hbm, lts_hbm, ids_hbm, ns_hbm, ts_hbm, out_hbm,
          ids_v, ts_v, win_v, gid_v, occ_v, prevts_v, alpha_v,
          ns_v, old_v, nb_v, sem_ids, sem_ts, sem_pts, sem_ns, sem_old,
          sem_cp, sem_cp2):
  w = lax.axis_index("s") * NC + lax.axis_index("c")
  lo = w * RW
  hi = jnp.minimum(lo + RW, N)
  base = jnp.minimum(lo, N - PREV)

  iota16 = lax.iota(jnp.int32, 16)
  last_lane = iota16 == 15
  rot1 = (iota16 + 1) & 15

  off2 = lo + jnp.where(w < NW - 1, RWL, 0)
  dcp = pltpu.async_copy(mem_hbm.at[pl.ds(lo, RWL)],
                         out_hbm.at[pl.ds(lo, RWL)], sem_cp)
  dcp2 = pltpu.async_copy(mem_hbm.at[pl.ds(off2, TL)],
                          out_hbm.at[pl.ds(off2, TL)], sem_cp2)
  dids = pltpu.async_copy(ids_hbm, ids_v, sem_ids)
  dts = pltpu.async_copy(ts_hbm, ts_v, sem_ts)
  dpts = pltpu.async_copy(lts_hbm.at[pl.ds(base, PREV)], prevts_v, sem_pts)

  @plsc.parallel_loop(0, NWV, step=1, unroll=4)
  def _init(i):
    win_v[pl.ds(i * 16, 16)] = jnp.full((16,), -1, jnp.int32)

  dids.wait()

  def scan_body(i, carry):
    for u in range(UNR):
      iv = i * UNR + u
      idv = ids_v[pl.ds(iv * 16, 16)]
      occv = iv * 16 + iota16
      valid = (idv >= lo) & (idv < hi)
      c = jnp.where(valid, ((idv - lo) << 14) | occv, -1)
      cs, _ = plsc.sort_key_val(c, c)
      nb_v[pl.ds(u * 16, 16)] = cs
      nxt = plsc.load_gather(nb_v, [u * 16 + rot1])
      lid = cs >> 14
      winner = (cs >= 0) & ((lid != (nxt >> 14)) | last_lane)
      plsc.store_scatter(win_v, [lid], cs & 0x3FFF, mask=winner)
    return carry
  lax.fori_loop(0, B // (16 * UNR), scan_body, 0)

  def comp_body(i, cnt):
    v = win_v[pl.ds(i * 16, 16)]
    m = v >= 0
    gidv = (i * 16 + iota16) + lo
    pref = plsc.cumsum(jnp.where(m, 1, 0).astype(jnp.int32))
    pos = jnp.minimum(cnt + pref - 1, CAP - 1)
    plsc.store_scatter(gid_v, [pos], gidv, mask=m)
    plsc.store_scatter(occ_v, [pos], v, mask=m)
    return cnt + pref[15]
  K = lax.fori_loop(0, NWV, comp_body, jnp.int32(0))

  npad = (-K) % CH
  kp = K + npad
  last = jnp.broadcast_to(jnp.maximum(K - 1, 0), (16,))
  lastg = plsc.load_gather(gid_v, [last])
  lasto = plsc.load_gather(occ_v, [last])
  for j in range(CH // 16):
    pidx = K + j * 16 + iota16
    pm = pidx < kp
    pidxc = jnp.minimum(pidx, CAP - 1)
    plsc.store_scatter(gid_v, [pidxc], lastg, mask=pm)
    plsc.store_scatter(occ_v, [pidxc], lasto, mask=pm)

  dts.wait()
  dpts.wait()
  dcp.wait()
  dcp2.wait()

  def chunk_body(cidx, carry):
    off = cidx * CH
    dns = pltpu.async_copy(ns_hbm.at[occ_v.at[pl.ds(off, CH)]], ns_v, sem_ns)
    dold = pltpu.async_copy(mem_hbm.at[gid_v.at[pl.ds(off, CH)]], old_v,
                            sem_old)
    for j in range(CH // 16):
      occv = occ_v[pl.ds(off + j * 16, 16)]
      gv = gid_v[pl.ds(off + j * 16, 16)]
      t = plsc.load_gather(ts_v, [occv])
      prev = plsc.load_gather(prevts_v, [gv - base])
      dt = jnp.maximum(t - prev, 0.0)
      alpha_v[pl.ds(j * 16, 16)] = jnp.exp(dt * COEF)
    dns.wait()
    dold.wait()

    @plsc.parallel_loop(0, CH, step=1, unroll=2)
    def _blend(r):
      av = plsc.load_gather(alpha_v, [jnp.broadcast_to(r, (16,))])
      for jj in range(D // 16):
        o = old_v[r, pl.ds(jj * 16, 16)]
        n = ns_v[r, pl.ds(jj * 16, 16)]
        old_v[r, pl.ds(jj * 16, 16)] = av * (o - n) + n

    pltpu.sync_copy(old_v, out_hbm.at[gid_v.at[pl.ds(off, CH)]])
    return carry
  lax.fori_loop(0, kp // CH, chunk_body, 0)


@jax.jit
def _ema_scatter(memory, last_update_ts, node_ids, new_states, ts):
  mesh = plsc.VectorSubcoreMesh(core_axis_name="c", subcore_axis_name="s")
  return pl.kernel(
      _body,
      out_type=jax.ShapeDtypeStruct((N, D), jnp.float32),
      mesh=mesh,
      compiler_params=pltpu.CompilerParams(needs_layout_passes=False),
      scratch_types=[
          pltpu.VMEM((B,), jnp.int32),
          pltpu.VMEM((B,), jnp.float32),
          pltpu.VMEM((NWV * 16,), jnp.int32),
          pltpu.VMEM((CAP,), jnp.int32),
          pltpu.VMEM((CAP,), jnp.int32),
          pltpu.VMEM((PREV,), jnp.float32),
          pltpu.VMEM((CH,), jnp.float32),
          pltpu.VMEM((CH, D), jnp.float32),
          pltpu.VMEM((CH, D), jnp.float32),
          pltpu.VMEM((UNR * 16,), jnp.int32),
          pltpu.SemaphoreType.DMA,
          pltpu.SemaphoreType.DMA,
          pltpu.SemaphoreType.DMA,
          pltpu.SemaphoreType.DMA,
          pltpu.SemaphoreType.DMA,
          pltpu.SemaphoreType.DMA,
          pltpu.SemaphoreType.DMA,
      ],
  )(memory, last_update_ts, node_ids, new_states, ts)


def kernel(memory, last_update_ts, node_ids, new_states, ts):
  return _ema_scatter(memory, last_update_ts, node_ids.astype(jnp.int32),
                      new_states, ts)


# speedup vs baseline: 11.1100x; 11.1100x over previous
import jax
import jax.numpy as jnp
from jax import lax
from jax.experimental import pallas as pl
from jax.experimental.pallas import tpu as pltpu
from jax.experimental.pallas import tpu_sc as plsc

N = 100000
D = 128
B = 16384
HALF_LIFE = 40.0
LN2 = 0.69314718

NC = 2
NS = 16
NW = NC * NS
RW = 3128
RWL = N - (NW - 1) * RW
TL = RW - RWL
PREV = 3136
CH = 128
CAP = 3200
NWV = PREV // 16
COEF = -LN2 / HALF_LIFE
UNR = 4


def _body(mem_hbm, lts_hbm, ids_hbm, ns_hbm, ts_hbm, out_hbm, mask_hbm,
          ids_v, ts_v, win_v, gid_v, occ_v, prevts_v, alpha_v,
          ns_v, old_v, nb_v, mask_v, sem_ids, sem_ts, sem_pts, sem_ns,
          sem_old, sem_cp, sem_cp2):
  w = lax.axis_index("s") * NC + lax.axis_index("c")
  lo = w * RW
  hi = jnp.minimum(lo + RW, N)
  base = jnp.minimum(lo, N - PREV)

  iota16 = lax.iota(jnp.int32, 16)
  last_lane = iota16 == 15
  rot1 = (iota16 + 1) & 15

  dids = pltpu.async_copy(ids_hbm, ids_v, sem_ids)
  dts = pltpu.async_copy(ts_hbm, ts_v, sem_ts)
  dpts = pltpu.async_copy(lts_hbm.at[pl.ds(base, PREV)], prevts_v, sem_pts)

  @plsc.parallel_loop(0, NWV, step=1, unroll=4)
  def _init(i):
    win_v[pl.ds(i * 16, 16)] = jnp.full((16,), -1, jnp.int32)

  dids.wait()

  def scan_body(i, carry):
    for u in range(UNR):
      iv = i * UNR + u
      idv = ids_v[pl.ds(iv * 16, 16)]
      occv = iv * 16 + iota16
      valid = (idv >= lo) & (idv < hi)
      c = jnp.where(valid, ((idv - lo) << 14) | occv, -1)
      cs, _ = plsc.sort_key_val(c, c)
      nb_v[pl.ds(u * 16, 16)] = cs
      nxt = plsc.load_gather(nb_v, [u * 16 + rot1])
      lid = cs >> 14
      winner = (cs >= 0) & ((lid != (nxt >> 14)) | last_lane)
      plsc.store_scatter(win_v, [lid], cs & 0x3FFF, mask=winner)
    return carry
  lax.fori_loop(0, B // (16 * UNR), scan_body, 0)

  @plsc.parallel_loop(0, NWV, step=1, unroll=4)
  def _mask(i):
    mask_v[pl.ds(i * 16, 16)] = jnp.where(
        win_v[pl.ds(i * 16, 16)] >= 0, 1, 0).astype(jnp.int32)

  off2 = lo + jnp.where(w < NW - 1, RWL, 0)
  src2 = jnp.where(w < NW - 1, RWL, 0)
  dmk = pltpu.async_copy(mask_v.at[pl.ds(0, RWL)],
                         mask_hbm.at[pl.ds(lo, RWL)], sem_cp)
  dmk2 = pltpu.async_copy(mask_v.at[pl.ds(src2, TL)],
                          mask_hbm.at[pl.ds(off2, TL)], sem_cp2)

  def comp_body(i, cnt):
    v = win_v[pl.ds(i * 16, 16)]
    m = v >= 0
    gidv = (i * 16 + iota16) + lo
    pref = plsc.cumsum(jnp.where(m, 1, 0).astype(jnp.int32))
    pos = jnp.minimum(cnt + pref - 1, CAP - 1)
    plsc.store_scatter(gid_v, [pos], gidv, mask=m)
    plsc.store_scatter(occ_v, [pos], v, mask=m)
    return cnt + pref[15]
  K = lax.fori_loop(0, NWV, comp_body, jnp.int32(0))

  npad = (-K) % CH
  kp = K + npad
  last = jnp.broadcast_to(jnp.maximum(K - 1, 0), (16,))
  lastg = plsc.load_gather(gid_v, [last])
  lasto = plsc.load_gather(occ_v, [last])
  for j in range(CH // 16):
    pidx = K + j * 16 + iota16
    pm = pidx < kp
    pidxc = jnp.minimum(pidx, CAP - 1)
    plsc.store_scatter(gid_v, [pidxc], lastg, mask=pm)
    plsc.store_scatter(occ_v, [pidxc], lasto, mask=pm)

  dts.wait()
  dpts.wait()

  def chunk_body(cidx, carry):
    off = cidx * CH
    dns = pltpu.async_copy(ns_hbm.at[occ_v.at[pl.ds(off, CH)]], ns_v, sem_ns)
    dold = pltpu.async_copy(mem_hbm.at[gid_v.at[pl.ds(off, CH)]], old_v,
                            sem_old)
    for j in range(CH // 16):
      occv = occ_v[pl.ds(off + j * 16, 16)]
      gv = gid_v[pl.ds(off + j * 16, 16)]
      t = plsc.load_gather(ts_v, [occv])
      prev = plsc.load_gather(prevts_v, [gv - base])
      dt = jnp.maximum(t - prev, 0.0)
      alpha_v[pl.ds(j * 16, 16)] = jnp.exp(dt * COEF)
    dns.wait()
    dold.wait()

    @plsc.parallel_loop(0, CH, step=1, unroll=2)
    def _blend(r):
      av = plsc.load_gather(alpha_v, [jnp.broadcast_to(r, (16,))])
      for jj in range(D // 16):
        o = old_v[r, pl.ds(jj * 16, 16)]
        n = ns_v[r, pl.ds(jj * 16, 16)]
        old_v[r, pl.ds(jj * 16, 16)] = av * (o - n) + n

    pltpu.sync_copy(old_v, out_hbm.at[gid_v.at[pl.ds(off, CH)]])
    return carry
  lax.fori_loop(0, kp // CH, chunk_body, 0)
  dmk.wait()
  dmk2.wait()


@jax.jit
def _ema_scatter(memory, last_update_ts, node_ids, new_states, ts):
  mesh = plsc.VectorSubcoreMesh(core_axis_name="c", subcore_axis_name="s")
  table, mask = pl.kernel(
      _body,
      out_type=[jax.ShapeDtypeStruct((N, D), jnp.float32),
                jax.ShapeDtypeStruct((N,), jnp.int32)],
      mesh=mesh,
      compiler_params=pltpu.CompilerParams(needs_layout_passes=False),
      scratch_types=[
          pltpu.VMEM((B,), jnp.int32),
          pltpu.VMEM((B,), jnp.float32),
          pltpu.VMEM((NWV * 16,), jnp.int32),
          pltpu.VMEM((CAP,), jnp.int32),
          pltpu.VMEM((CAP,), jnp.int32),
          pltpu.VMEM((PREV,), jnp.float32),
          pltpu.VMEM((CH,), jnp.float32),
          pltpu.VMEM((CH, D), jnp.float32),
          pltpu.VMEM((CH, D), jnp.float32),
          pltpu.VMEM((UNR * 16,), jnp.int32),
          pltpu.VMEM((PREV,), jnp.int32),
          pltpu.SemaphoreType.DMA,
          pltpu.SemaphoreType.DMA,
          pltpu.SemaphoreType.DMA,
          pltpu.SemaphoreType.DMA,
          pltpu.SemaphoreType.DMA,
          pltpu.SemaphoreType.DMA,
          pltpu.SemaphoreType.DMA,
      ],
  )(memory, last_update_ts, node_ids, new_states, ts)
  return jnp.where(mask[:, None] != 0, table, memory)


def kernel(memory, last_update_ts, node_ids, new_states, ts):
  return _ema_scatter(memory, last_update_ts, node_ids.astype(jnp.int32),
                      new_states, ts)
